# baseline (speedup 1.0000x reference)
import functools

import jax
import jax.numpy as jnp
import numpy as np
from jax import lax
from jax.experimental import pallas as pl
from jax.experimental.pallas import tpu as pltpu
from jax.experimental.pallas import tpu_sc as plsc

BATCH = 512
N_LOGITS = 33416
Y = 1024
X = 64
GRID_WORDS = Y * X
PERIOD_ROWS = 63
PERIOD_IN = 2079
PERIOD_OUT = PERIOD_ROWS * X
NUM_PERIODS = 16
TAIL_ROWS = 16
IN_PAD = N_LOGITS + 24

NC = 2
NS = 16
NW = NC * NS
SAMPLES_PER_WORKER = BATCH // NW


def _period_chunks(rows):
  chunks = []
  for r in range(rows):
    n = r + 2
    s = r * (r + 3) // 2
    for c in range(0, n, 16):
      chunks.append((s + c, r * X + c, min(16, n - c)))
  return chunks


def _emit_period(in_ref, grid_ref, ibase, obase, rows, group=10):
  iota = lax.iota(jnp.int32, 16)
  ninf = jnp.float32(-jnp.inf)
  chunks = _period_chunks(rows)
  for g0 in range(0, len(chunks), group):
    grp = chunks[g0:g0 + group]
    vals = [in_ref[pl.ds(ibase + so, 16)] for so, _, _ in grp]
    vals = [v if rem == 16 else jnp.where(iota < rem, v, ninf)
            for v, (_, _, rem) in zip(vals, grp)]
    for v, (_, do, _) in zip(vals, grp):
      grid_ref[pl.ds(obase + do, 16)] = v


def _sc_body(logits_hbm, out_hbm, in_v, grid_v):
  c = lax.axis_index("c")
  s = lax.axis_index("s")
  wid = s * NC + c

  ninf_vec = jnp.full((16,), -jnp.inf, jnp.float32)

  def init_body(i, _):
    grid_v[pl.ds(i * 16, 16)] = ninf_vec
    return _

  lax.fori_loop(0, GRID_WORDS // 16, init_body, None)

  def sample_body(i, _):
    b = wid * SAMPLES_PER_WORKER + i
    pltpu.sync_copy(logits_hbm.at[pl.ds(b * N_LOGITS, N_LOGITS)],
                    in_v.at[pl.ds(0, N_LOGITS)])

    def period_body(k, _):
      _emit_period(in_v, grid_v, k * PERIOD_IN, k * PERIOD_OUT, PERIOD_ROWS)
      return _

    lax.fori_loop(0, NUM_PERIODS, period_body, None)
    _emit_period(in_v, grid_v, NUM_PERIODS * PERIOD_IN,
                 NUM_PERIODS * PERIOD_OUT, TAIL_ROWS)
    pltpu.sync_copy(grid_v, out_hbm.at[pl.ds(b * GRID_WORDS, GRID_WORDS)])
    return _

  lax.fori_loop(0, SAMPLES_PER_WORKER, sample_body, None)


def kernel(logits):
  mesh = plsc.VectorSubcoreMesh(core_axis_name="c", subcore_axis_name="s")
  run = pl.kernel(
      _sc_body,
      out_type=jax.ShapeDtypeStruct((BATCH * GRID_WORDS,), jnp.float32),
      mesh=mesh,
      scratch_types=[
          pltpu.VMEM((IN_PAD,), jnp.float32),
          pltpu.VMEM((GRID_WORDS,), jnp.float32),
      ],
  )
  flat = run(logits.reshape(BATCH * N_LOGITS))
  return flat.reshape(BATCH, Y, X)


# speedup vs baseline: 3.0877x; 3.0877x over previous
import jax
import jax.numpy as jnp
import numpy as np
from jax import lax
from jax.experimental import pallas as pl
from jax.experimental.pallas import tpu as pltpu
from jax.experimental.pallas import tpu_sc as plsc

BATCH = 512
N_LOGITS = 33416
Y = 1024
X = 64

_NVEC = (np.arange(Y) % 63) + 2
_START = np.concatenate([[0], np.cumsum(_NVEC)])

_TAB = np.repeat(np.arange(Y) * X, _NVEC) + (
    np.arange(N_LOGITS) - np.repeat(_START[:-1], _NVEC))
_TAB_PAD = 16 * ((N_LOGITS + 15) // 16)
_TAB_FULL = np.zeros(_TAB_PAD, np.int32)
_TAB_FULL[:N_LOGITS] = _TAB

_NPIECES = Y // 128
_YW = 128
_LO = [int(_START[_YW * k]) for k in range(_NPIECES)]
_HI = [int(_START[_YW * (k + 1)]) for k in range(_NPIECES)]
_W0 = [128 * (_LO[k] // 128) for k in range(_NPIECES)]
_FULL_TILES = 128 * (N_LOGITS // 128)
_WIDTH = [128 * ((_HI[k] - _W0[k] + 127) // 128) for k in range(_NPIECES)]
_WIDTH[-1] = _FULL_TILES - _W0[-1]
_STAGE_COLS = max(_WIDTH)

NC = 2
NS = 16
NW = NC * NS
GROUPS = BATCH // 8
GROUPS_PER_WORKER = GROUPS // NW


def _sc_body(logits_hbm, tab_hbm, out_hbm, stage_v, tab_v, piece_v0,
             piece_v1, strag_v, sem0, sem1):
  c = lax.axis_index("c")
  s = lax.axis_index("s")
  wid = s * NC + c

  pltpu.sync_copy(tab_hbm, tab_v)
  ninf_vec = jnp.full((16,), -jnp.inf, jnp.float32)

  def init_piece(piece_v):
    def ib(r, _):
      for cc in range(0, _YW, 16):
        piece_v[r, pl.ds(cc, 16)] = ninf_vec
      return _
    lax.fori_loop(0, X, ib, None)

  def group_body(gi, _):
    g = wid + NW * gi
    for k in range(_NPIECES):
      lo, hi, w0 = _LO[k], _HI[k], _W0[k]
      width = _WIDTH[k]
      pltpu.sync_copy(logits_hbm.at[pl.ds(g * 8, 8), pl.ds(w0, width)],
                      stage_v.at[:, pl.ds(0, width)])
      if k == _NPIECES - 1:
        pltpu.sync_copy(
            logits_hbm.at[pl.ds(g * 8, 8), pl.ds(_FULL_TILES, 8)], strag_v)
      j_lo, j_hi = lo // 16, (hi + 15) // 16
      ybase = jnp.int32(_YW * k)

      def chunk(piece_v, sample, j, lane_lo=None, lane_hi=None,
                from_strag=False):
        if from_strag:
          lane = lax.iota(jnp.int32, 16)
          v = plsc.load_gather(strag_v, [jnp.broadcast_to(sample, (16,)),
                                         lane & 7])
        else:
          v = stage_v[sample, pl.ds(j * 16 - w0, 16)]
        d = tab_v[pl.ds(j * 16, 16)]
        row = d & 63
        col = (d >> 6) - ybase
        if lane_lo is None and lane_hi is None:
          plsc.store_scatter(piece_v, [row, col], v)
        else:
          lane = lax.iota(jnp.int32, 16)
          mask = (lane >= lane_lo) if lane_lo is not None else (lane < lane_hi)
          plsc.store_scatter(piece_v, [row, col], v, mask=mask)

      def wait_piece(piece_v, sem):
        pltpu.make_async_copy(
            piece_v, out_hbm.at[0, :, pl.ds(0, _YW)], sem).wait()

      def process(piece_v, sem, sample, first):
        b = g * 8 + sample
        if first == 'maybe':
          @pl.when(gi > 0)
          def _w():
            wait_piece(piece_v, sem)
        elif not first:
          wait_piece(piece_v, sem)
        init_piece(piece_v)
        if lo % 16 != 0:
          chunk(piece_v, sample, j_lo, lane_lo=lo - j_lo * 16)
          jl = j_lo + 1
        else:
          jl = j_lo
        if hi % 16 != 0:
          chunk(piece_v, sample, j_hi - 1, lane_hi=hi - 16 * (j_hi - 1),
                from_strag=(k == _NPIECES - 1))
          jh = j_hi - 1
        else:
          jh = j_hi

        G = 8
        nmain = (jh - jl) // G * G

        def jgroup(t, _):
          j0 = (jl + t * G) * 16
          ds_ = [tab_v[pl.ds(j0 + 16 * i, 16)] for i in range(G)]
          vs = [stage_v[sample, pl.ds(j0 + (16 * i - w0), 16)]
                for i in range(G)]
          for i in range(G):
            d = ds_[i]
            plsc.store_scatter(piece_v, [d & 63, (d >> 6) - ybase], vs[i])
          return _

        lax.fori_loop(0, nmain // G, jgroup, None)
        for j in range(jl + nmain, jh):
          chunk(piece_v, sample, j)
        pltpu.async_copy(piece_v, out_hbm.at[b, :, pl.ds(_YW * k, _YW)], sem)

      def pair_body(t, _):
        process(piece_v0, sem0, 2 * t, False)
        process(piece_v1, sem1, 2 * t + 1, False)
        return _

      first01 = 'maybe' if k == 0 else False
      process(piece_v0, sem0, 0, first01)
      process(piece_v1, sem1, 1, first01)
      lax.fori_loop(1, 4, pair_body, None)
    return _

  lax.fori_loop(0, GROUPS_PER_WORKER, group_body, None)
  for pv, sm in ((piece_v0, sem0), (piece_v1, sem1)):
    pltpu.make_async_copy(pv, out_hbm.at[0, :, pl.ds(0, _YW)], sm).wait()


def kernel(logits):
  mesh = plsc.VectorSubcoreMesh(core_axis_name="c", subcore_axis_name="s")
  run = pl.kernel(
      _sc_body,
      out_type=jax.ShapeDtypeStruct((BATCH, X, Y), jnp.float32),
      mesh=mesh,
      scratch_types=[
          pltpu.VMEM((8, _STAGE_COLS), jnp.float32),
          pltpu.VMEM((_TAB_PAD,), jnp.int32),
          pltpu.VMEM((X, _YW), jnp.float32),
          pltpu.VMEM((X, _YW), jnp.float32),
          pltpu.VMEM((8, 8), jnp.float32),
          pltpu.SemaphoreType.DMA,
          pltpu.SemaphoreType.DMA,
      ],
      compiler_params=pltpu.CompilerParams(
          use_tc_tiling_on_sc=True, needs_layout_passes=False),
  )
  return run(logits, jnp.asarray(_TAB_FULL)).transpose(0, 2, 1)
